# Initial kernel scaffold; baseline (speedup 1.0000x reference)
import functools

import jax
import jax.numpy as jnp
from jax import lax
from jax.experimental import pallas as pl
from jax.experimental.pallas import tpu as pltpu
from jax.experimental.pallas import tpu_sc as plsc

N = 10000
E = 160000
D = 256
H = 128
C = 64
K = 10
ALPHA = 0.1
BETA = 1.0 - ALPHA

NC = 2
NS = 16
NW = NC * NS
RPT = 320
NPAD = NW * RPT
EB = 128
ET = E + NW * EB
SLAB = NS * RPT
PADROW = SLAB
AGGR = SLAB + 16
RB = 16

_MESH = plsc.VectorSubcoreMesh(core_axis_name="c", subcore_axis_name="s")


def _prep(edge_index):
    s = edge_index[0].astype(jnp.int32)
    d = edge_index[1].astype(jnp.int32)
    order = jnp.argsort(d)
    ss = jnp.take(s, order)
    ds = jnp.take(d, order)
    starts = jnp.searchsorted(
        ds, (RPT * jnp.arange(NW + 1)).astype(jnp.int32)).astype(jnp.int32)
    cnt = starts[1:] - starts[:-1]
    cntp = ((cnt + EB - 1) // EB) * EB
    astart = jnp.concatenate(
        [jnp.zeros((1,), jnp.int32), jnp.cumsum(cntp).astype(jnp.int32)])
    bucket = ds // RPT
    rank = jnp.arange(E, dtype=jnp.int32) - starts[bucket]
    pos = astart[bucket] + rank
    ss_pad = jnp.zeros((ET,), jnp.int32).at[pos].set(ss)
    dloc = (ds % RPT) + RPT * (bucket % NS)
    dloc_pad = jnp.full((ET,), PADROW, jnp.int32).at[pos].set(dloc)
    nb = cntp // EB
    meta = jnp.zeros((NW, 16), jnp.int32)
    meta = meta.at[:, 0].set(astart[:-1]).at[:, 1].set(nb)
    return ss_pad, dloc_pad, meta


@functools.partial(
    pl.kernel,
    out_type=jax.ShapeDtypeStruct((NPAD,), jnp.float32),
    mesh=_MESH,
    scratch_types=[
        pltpu.VMEM((16,), jnp.int32),
        pltpu.VMEM((EB,), jnp.int32),
        pltpu.VMEM((AGGR,), jnp.float32),
        pltpu.VMEM((RPT,), jnp.float32),
    ],
)
def _deg_kernel(dloc_hbm, meta_hbm, deg_hbm, meta_v, didx_v, cnt_v, dbuf_v):
    c = lax.axis_index("c")
    sb = lax.axis_index("s")
    wid = c * NS + sb
    pltpu.sync_copy(meta_hbm.at[wid], meta_v)
    mv = meta_v[...]
    lanes = lax.iota(jnp.int32, 16)
    e0 = jnp.max(jnp.where(lanes == 0, mv, 0))
    nb = jnp.max(jnp.where(lanes == 1, mv, 0))

    zeros16 = jnp.zeros((16,), jnp.float32)
    ones16 = jnp.ones((16,), jnp.float32)

    def zero_body(i, carry):
        cnt_v[pl.ds(i * 16, 16)] = zeros16
        return carry

    lax.fori_loop(0, AGGR // 16, zero_body, 0)

    def batch_body(b, carry):
        base = pl.multiple_of(e0 + b * EB, EB)
        pltpu.sync_copy(dloc_hbm.at[pl.ds(base, EB)], didx_v)
        for j in range(EB // 16):
            idx = didx_v[pl.ds(j * 16, 16)] - sb * RPT
            plsc.addupdate_scatter(cnt_v, [idx], ones16)
        return carry

    lax.fori_loop(0, nb, batch_body, 0)

    def deg_body(i, carry):
        dbuf_v[pl.ds(i * 16, 16)] = cnt_v[pl.ds(i * 16, 16)] + 1.0
        return carry

    lax.fori_loop(0, RPT // 16, deg_body, 0)
    pltpu.sync_copy(dbuf_v, deg_hbm.at[pl.ds(wid * RPT, RPT)])


@functools.partial(
    pl.kernel,
    out_type=jax.ShapeDtypeStruct((NPAD, H), jnp.float32),
    mesh=_MESH,
    scratch_types=[
        pltpu.VMEM((16,), jnp.int32),
        pltpu.VMEM((EB,), jnp.int32),
        pltpu.VMEM((EB,), jnp.int32),
        pltpu.VMEM((EB, H), jnp.float32),
        pltpu.VMEM((RB, H), jnp.float32),
        pltpu.VMEM((RB, H), jnp.float32),
        pltpu.VMEM((RB, H), jnp.float32),
        pltpu.VMEM((RB, H), jnp.float32),
        pltpu.VMEM_SHARED((SLAB + 16, H), jnp.float32),
        pltpu.SemaphoreType.DMA,
    ],
)
def _hop_kernel(y_hbm, p_hbm, q_hbm, ss_hbm, dloc_hbm, meta_hbm, zslab_hbm,
                ynew_hbm,
                meta_v, sidx_v, didx_v, rows_v, acc_v, yv, pv, qv,
                agg_sh, gsem):
    c = lax.axis_index("c")
    sb = lax.axis_index("s")
    wid = c * NS + sb
    pltpu.sync_copy(meta_hbm.at[wid], meta_v)
    mv = meta_v[...]
    lanes = lax.iota(jnp.int32, 16)
    e0 = jnp.max(jnp.where(lanes == 0, mv, 0))
    nb = jnp.max(jnp.where(lanes == 1, mv, 0))
    slab0 = sb * RPT

    pltpu.sync_copy(zslab_hbm, agg_sh.at[pl.ds(slab0, RPT)])

    def batch_body(b, carry):
        base = pl.multiple_of(e0 + b * EB, EB)
        pltpu.sync_copy(ss_hbm.at[pl.ds(base, EB)], sidx_v)
        pltpu.sync_copy(dloc_hbm.at[pl.ds(base, EB)], didx_v)
        pltpu.async_copy(y_hbm.at[sidx_v], rows_v, gsem).wait()
        pltpu.sync_copy(rows_v, agg_sh.at[didx_v], add=True)
        return carry

    lax.fori_loop(0, nb, batch_body, 0)

    def row_body(i, carry):
        r0 = slab0 + i * RB
        g0 = wid * RPT + i * RB
        pltpu.sync_copy(agg_sh.at[pl.ds(r0, RB)], acc_v)
        pltpu.sync_copy(y_hbm.at[pl.ds(g0, RB)], yv)
        pltpu.sync_copy(p_hbm.at[pl.ds(g0, RB)], pv)
        pltpu.sync_copy(q_hbm.at[pl.ds(g0, RB)], qv)
        for r in range(RB):
            for j in range(H // 16):
                sl = pl.ds(j * 16, 16)
                a = acc_v[r, sl] + yv[r, sl]
                yv[r, sl] = pv[r, sl] + qv[r, sl] * a
        pltpu.sync_copy(yv, ynew_hbm.at[pl.ds(g0, RB)])
        return carry

    lax.fori_loop(0, RPT // RB, row_body, 0)


_TCM = 1024


def _tc1_body(x_ref, w_ref, b_ref, degb_ref, y0_ref, p_ref, q_ref, dinv_ref):
    h = jnp.dot(x_ref[...], w_ref[...],
                preferred_element_type=jnp.float32) + b_ref[...]
    dinv = lax.rsqrt(degb_ref[...])
    y0 = dinv * h
    y0_ref[...] = y0
    p_ref[...] = ALPHA * y0
    q_ref[...] = BETA * dinv * dinv
    dinv_ref[...] = dinv


def _tc1(x_pad, W1, b1, degb):
    grid = (NPAD // _TCM,)
    fo = jax.ShapeDtypeStruct((NPAD, H), jnp.float32)
    return pl.pallas_call(
        _tc1_body,
        grid=grid,
        in_specs=[
            pl.BlockSpec((_TCM, D), lambda i: (i, 0)),
            pl.BlockSpec((D, H), lambda i: (0, 0)),
            pl.BlockSpec((1, H), lambda i: (0, 0)),
            pl.BlockSpec((_TCM, H), lambda i: (i, 0)),
        ],
        out_specs=[pl.BlockSpec((_TCM, H), lambda i: (i, 0))] * 4,
        out_shape=[fo, fo, fo, fo],
    )(x_pad, W1, b1, degb)


def _tc2_body(y_ref, dinv_ref, w_ref, b_ref, g_ref, be_ref, m_ref, v_ref,
              y0_ref, p_ref):
    dinv = dinv_ref[...]
    xx = y_ref[...] / dinv
    t = (xx - m_ref[...]) * lax.rsqrt(v_ref[...] + 1e-5) * g_ref[...] \
        + be_ref[...]
    t = jnp.maximum(t, 0.0)
    h2 = jnp.dot(t, w_ref[...], preferred_element_type=jnp.float32) \
        + b_ref[...]
    y0 = dinv * h2
    y0_ref[...] = y0
    p_ref[...] = ALPHA * y0


def _tc2(y, dinvb, W2, b2, g1, be1, m1, v1):
    grid = (NPAD // _TCM,)
    fo = jax.ShapeDtypeStruct((NPAD, H), jnp.float32)
    vec = pl.BlockSpec((1, H), lambda i: (0, 0))
    return pl.pallas_call(
        _tc2_body,
        grid=grid,
        in_specs=[
            pl.BlockSpec((_TCM, H), lambda i: (i, 0)),
            pl.BlockSpec((_TCM, H), lambda i: (i, 0)),
            pl.BlockSpec((H, H), lambda i: (0, 0)),
            vec, vec, vec, vec, vec,
        ],
        out_specs=[pl.BlockSpec((_TCM, H), lambda i: (i, 0))] * 2,
        out_shape=[fo, fo],
    )(y, dinvb, W2, b2, g1, be1, m1, v1)


def _tc3_body(y_ref, dinv_ref, w_ref, b_ref, g_ref, be_ref, m_ref, v_ref,
              out_ref):
    xx = y_ref[...] / dinv_ref[...]
    t = (xx - m_ref[...]) * lax.rsqrt(v_ref[...] + 1e-5) * g_ref[...] \
        + be_ref[...]
    t = jnp.maximum(t, 0.0)
    out_ref[...] = jnp.dot(t, w_ref[...],
                           preferred_element_type=jnp.float32) + b_ref[...]


def _tc3(y, dinvb, W3, b3, g2, be2, m2, v2):
    grid = (NPAD // _TCM,)
    vec = pl.BlockSpec((1, H), lambda i: (0, 0))
    return pl.pallas_call(
        _tc3_body,
        grid=grid,
        in_specs=[
            pl.BlockSpec((_TCM, H), lambda i: (i, 0)),
            pl.BlockSpec((_TCM, H), lambda i: (i, 0)),
            pl.BlockSpec((H, C), lambda i: (0, 0)),
            pl.BlockSpec((1, C), lambda i: (0, 0)),
            vec, vec, vec, vec,
        ],
        out_specs=pl.BlockSpec((_TCM, C), lambda i: (i, 0)),
        out_shape=jax.ShapeDtypeStruct((NPAD, C), jnp.float32),
    )(y, dinvb, W3, b3, g2, be2, m2, v2)


def kernel(x, edge_index, W1, b1, W2, b2, W3, b3,
           g1, be1, m1, v1, g2, be2, m2, v2):
    ss_pad, dloc_pad, meta = _prep(edge_index)
    x_pad = jnp.zeros((NPAD, D), jnp.float32).at[:N].set(x)
    zslab = jnp.zeros((RPT, H), jnp.float32)

    deg = _deg_kernel(dloc_pad, meta)
    degb = jnp.broadcast_to(deg[:, None], (NPAD, H))

    y, p, q, dinvb = _tc1(x_pad, W1, b1.reshape(1, H), degb)
    for _ in range(K):
        y = _hop_kernel(y, p, q, ss_pad, dloc_pad, meta, zslab)

    y, p = _tc2(y, dinvb, W2, b2.reshape(1, H),
                g1.reshape(1, H), be1.reshape(1, H),
                m1.reshape(1, H), v1.reshape(1, H))
    for _ in range(K):
        y = _hop_kernel(y, p, q, ss_pad, dloc_pad, meta, zslab)

    out = _tc3(y, dinvb, W3, b3.reshape(1, C),
               g2.reshape(1, H), be2.reshape(1, H),
               m2.reshape(1, H), v2.reshape(1, H))
    return out[:N]


# speedup vs baseline: 4.0158x; 4.0158x over previous
import functools

import jax
import jax.numpy as jnp
from jax import lax
from jax.experimental import pallas as pl
from jax.experimental.pallas import tpu as pltpu
from jax.experimental.pallas import tpu_sc as plsc

N = 10000
E = 160000
D = 256
H = 128
C = 64
K = 10
ALPHA = 0.1
BETA = 1.0 - ALPHA

NC = 2
NS = 16
NW = NC * NS
RPT = 320
NPAD = NW * RPT
EB = 128
ET = E + NW * EB
SLAB = NS * RPT
PADROW = SLAB
AGGR = SLAB + 16
RB = 16

def _prep(edge_index):
    s = edge_index[0].astype(jnp.int32)
    d = edge_index[1].astype(jnp.int32)
    order = jnp.argsort(d)
    ss = jnp.take(s, order)
    ds = jnp.take(d, order)
    starts = jnp.searchsorted(
        ds, (RPT * jnp.arange(NW + 1)).astype(jnp.int32)).astype(jnp.int32)
    cnt = starts[1:] - starts[:-1]
    cntp = ((cnt + EB - 1) // EB) * EB
    astart = jnp.concatenate(
        [jnp.zeros((1,), jnp.int32), jnp.cumsum(cntp).astype(jnp.int32)])
    bucket = ds // RPT
    rank = jnp.arange(E, dtype=jnp.int32) - starts[bucket]
    pos = astart[bucket] + rank
    ss_pad = jnp.zeros((ET,), jnp.int32).at[pos].set(ss)
    dloc = (ds % RPT) + RPT * (bucket % NS)
    dloc_pad = jnp.full((ET,), PADROW, jnp.int32).at[pos].set(dloc)
    nb = cntp // EB
    meta = jnp.zeros((NW, 16), jnp.int32)
    meta = meta.at[:, 0].set(astart[:-1]).at[:, 1].set(nb)
    return ss_pad, dloc_pad, meta


@functools.cache
def _make_deg_kernel():
    mesh = plsc.VectorSubcoreMesh(core_axis_name="c", subcore_axis_name="s")
    return functools.partial(
        pl.kernel,
        out_type=jax.ShapeDtypeStruct((NPAD, 16), jnp.float32),
        mesh=mesh,
        scratch_types=[
            pltpu.VMEM((16,), jnp.int32),
            pltpu.VMEM((EB,), jnp.int32),
            pltpu.VMEM((EB, 16), jnp.float32),
            pltpu.VMEM_SHARED((AGGR, 16), jnp.float32),
        ],
    )(_deg_body)


def _deg_body(dloc_hbm, meta_hbm, ones_hbm, zdeg_hbm, deg_hbm,
              meta_v, didx_v, ones_v, cnt_sh):
    c = lax.axis_index("c")
    sb = lax.axis_index("s")
    wid = c * NS + sb
    pltpu.sync_copy(meta_hbm.at[wid], meta_v)
    mv = meta_v[...]
    e0 = mv[0]
    nb = mv[1]
    slab0 = sb * RPT

    pltpu.sync_copy(ones_hbm, ones_v)
    pltpu.sync_copy(zdeg_hbm, cnt_sh.at[pl.ds(slab0, RPT)])

    def batch_body(b, carry):
        base = pl.multiple_of(e0 + b * EB, EB)
        pltpu.sync_copy(dloc_hbm.at[pl.ds(base, EB)], didx_v)
        pltpu.sync_copy(ones_v, cnt_sh.at[didx_v], add=True)
        return carry

    lax.fori_loop(0, nb, batch_body, 0)
    pltpu.sync_copy(cnt_sh.at[pl.ds(slab0, RPT)],
                    deg_hbm.at[pl.ds(wid * RPT, RPT)])


@functools.cache
def _make_hop_kernel():
    mesh = plsc.VectorSubcoreMesh(core_axis_name="c", subcore_axis_name="s")
    return functools.partial(
        pl.kernel,
        out_type=jax.ShapeDtypeStruct((NPAD, H), jnp.float32),
        mesh=mesh,
        scratch_types=[
            pltpu.VMEM((16,), jnp.int32),
            pltpu.VMEM((EB,), jnp.int32),
            pltpu.VMEM((EB,), jnp.int32),
            pltpu.VMEM((EB, H), jnp.float32),
            pltpu.VMEM((RB, H), jnp.float32),
            pltpu.VMEM((RB, H), jnp.float32),
            pltpu.VMEM((RB, H), jnp.float32),
            pltpu.VMEM((RB, H), jnp.float32),
            pltpu.VMEM_SHARED((AGGR, H), jnp.float32),
            pltpu.SemaphoreType.DMA,
        ],
    )(_hop_body)


def _hop_body(y_hbm, p_hbm, q_hbm, ss_hbm, dloc_hbm, meta_hbm, zslab_hbm,
              ynew_hbm,
              meta_v, sidx_v, didx_v, rows_v, acc_v, yv, pv, qv,
              agg_sh, gsem):
    c = lax.axis_index("c")
    sb = lax.axis_index("s")
    wid = c * NS + sb
    pltpu.sync_copy(meta_hbm.at[wid], meta_v)
    mv = meta_v[...]
    e0 = mv[0]
    nb = mv[1]
    slab0 = sb * RPT

    pltpu.sync_copy(zslab_hbm, agg_sh.at[pl.ds(slab0, RPT)])

    def batch_body(b, carry):
        base = pl.multiple_of(e0 + b * EB, EB)
        pltpu.sync_copy(ss_hbm.at[pl.ds(base, EB)], sidx_v)
        pltpu.sync_copy(dloc_hbm.at[pl.ds(base, EB)], didx_v)
        pltpu.async_copy(y_hbm.at[sidx_v], rows_v, gsem).wait()
        pltpu.sync_copy(rows_v, agg_sh.at[didx_v], add=True)
        return carry

    lax.fori_loop(0, nb, batch_body, 0)

    def row_body(i, carry):
        r0 = slab0 + i * RB
        g0 = wid * RPT + i * RB
        pltpu.sync_copy(agg_sh.at[pl.ds(r0, RB)], acc_v)
        pltpu.sync_copy(y_hbm.at[pl.ds(g0, RB)], yv)
        pltpu.sync_copy(p_hbm.at[pl.ds(g0, RB)], pv)
        pltpu.sync_copy(q_hbm.at[pl.ds(g0, RB)], qv)
        for r in range(RB):
            for j in range(H // 16):
                sl = pl.ds(j * 16, 16)
                a = acc_v[r, sl] + yv[r, sl]
                yv[r, sl] = pv[r, sl] + qv[r, sl] * a
        pltpu.sync_copy(yv, ynew_hbm.at[pl.ds(g0, RB)])
        return carry

    lax.fori_loop(0, RPT // RB, row_body, 0)


_TCM = 1024


def _tc1_body(x_ref, w_ref, b_ref, degb_ref, y0_ref, p_ref, q_ref, dinv_ref):
    h = jnp.dot(x_ref[...], w_ref[...],
                preferred_element_type=jnp.float32) + b_ref[...]
    dinv = lax.rsqrt(degb_ref[...])
    y0 = dinv * h
    y0_ref[...] = y0
    p_ref[...] = ALPHA * y0
    q_ref[...] = BETA * dinv * dinv
    dinv_ref[...] = dinv


def _tc1(x_pad, W1, b1, degb):
    grid = (NPAD // _TCM,)
    fo = jax.ShapeDtypeStruct((NPAD, H), jnp.float32)
    return pl.pallas_call(
        _tc1_body,
        grid=grid,
        in_specs=[
            pl.BlockSpec((_TCM, D), lambda i: (i, 0)),
            pl.BlockSpec((D, H), lambda i: (0, 0)),
            pl.BlockSpec((1, H), lambda i: (0, 0)),
            pl.BlockSpec((_TCM, H), lambda i: (i, 0)),
        ],
        out_specs=[pl.BlockSpec((_TCM, H), lambda i: (i, 0))] * 4,
        out_shape=[fo, fo, fo, fo],
    )(x_pad, W1, b1, degb)


def _tc2_body(y_ref, dinv_ref, w_ref, b_ref, g_ref, be_ref, m_ref, v_ref,
              y0_ref, p_ref):
    dinv = dinv_ref[...]
    xx = y_ref[...] / dinv
    t = (xx - m_ref[...]) * lax.rsqrt(v_ref[...] + 1e-5) * g_ref[...] \
        + be_ref[...]
    t = jnp.maximum(t, 0.0)
    h2 = jnp.dot(t, w_ref[...], preferred_element_type=jnp.float32) \
        + b_ref[...]
    y0 = dinv * h2
    y0_ref[...] = y0
    p_ref[...] = ALPHA * y0


def _tc2(y, dinvb, W2, b2, g1, be1, m1, v1):
    grid = (NPAD // _TCM,)
    fo = jax.ShapeDtypeStruct((NPAD, H), jnp.float32)
    vec = pl.BlockSpec((1, H), lambda i: (0, 0))
    return pl.pallas_call(
        _tc2_body,
        grid=grid,
        in_specs=[
            pl.BlockSpec((_TCM, H), lambda i: (i, 0)),
            pl.BlockSpec((_TCM, H), lambda i: (i, 0)),
            pl.BlockSpec((H, H), lambda i: (0, 0)),
            vec, vec, vec, vec, vec,
        ],
        out_specs=[pl.BlockSpec((_TCM, H), lambda i: (i, 0))] * 2,
        out_shape=[fo, fo],
    )(y, dinvb, W2, b2, g1, be1, m1, v1)


def _tc3_body(y_ref, dinv_ref, w_ref, b_ref, g_ref, be_ref, m_ref, v_ref,
              out_ref):
    xx = y_ref[...] / dinv_ref[...]
    t = (xx - m_ref[...]) * lax.rsqrt(v_ref[...] + 1e-5) * g_ref[...] \
        + be_ref[...]
    t = jnp.maximum(t, 0.0)
    out_ref[...] = jnp.dot(t, w_ref[...],
                           preferred_element_type=jnp.float32) + b_ref[...]


def _tc3(y, dinvb, W3, b3, g2, be2, m2, v2):
    grid = (NPAD // _TCM,)
    vec = pl.BlockSpec((1, H), lambda i: (0, 0))
    return pl.pallas_call(
        _tc3_body,
        grid=grid,
        in_specs=[
            pl.BlockSpec((_TCM, H), lambda i: (i, 0)),
            pl.BlockSpec((_TCM, H), lambda i: (i, 0)),
            pl.BlockSpec((H, C), lambda i: (0, 0)),
            pl.BlockSpec((1, C), lambda i: (0, 0)),
            vec, vec, vec, vec,
        ],
        out_specs=pl.BlockSpec((_TCM, C), lambda i: (i, 0)),
        out_shape=jax.ShapeDtypeStruct((NPAD, C), jnp.float32),
    )(y, dinvb, W3, b3, g2, be2, m2, v2)


def kernel(x, edge_index, W1, b1, W2, b2, W3, b3,
           g1, be1, m1, v1, g2, be2, m2, v2):
    ss_pad, dloc_pad, meta = _prep(edge_index)
    x_pad = jnp.zeros((NPAD, D), jnp.float32).at[:N].set(x)
    zslab = jnp.zeros((RPT, H), jnp.float32)

    ones_eb = jnp.ones((EB, 16), jnp.float32)
    zdeg = jnp.zeros((RPT, 16), jnp.float32)
    cnt16 = _make_deg_kernel()(dloc_pad, meta, ones_eb, zdeg)
    degb = jnp.broadcast_to(cnt16[:, :1] + 1.0, (NPAD, H))
    hop = _make_hop_kernel()

    y, p, q, dinvb = _tc1(x_pad, W1, b1.reshape(1, H), degb)
    for _ in range(K):
        y = hop(y, p, q, ss_pad, dloc_pad, meta, zslab)

    y, p = _tc2(y, dinvb, W2, b2.reshape(1, H),
                g1.reshape(1, H), be1.reshape(1, H),
                m1.reshape(1, H), v1.reshape(1, H))
    for _ in range(K):
        y = hop(y, p, q, ss_pad, dloc_pad, meta, zslab)

    out = _tc3(y, dinvb, W3, b3.reshape(1, C),
               g2.reshape(1, H), be2.reshape(1, H),
               m2.reshape(1, H), v2.reshape(1, H))
    return out[:N]
